# Initial kernel scaffold; baseline (speedup 1.0000x reference)
import jax
import jax.numpy as jnp
from jax.experimental import pallas as pl


def kernel(weights, offsets, ray_indices, num_rays):
    raise NotImplementedError("write your pallas kernel here")


# speedup vs baseline: 1.4262x; 1.4262x over previous
import functools

import jax
import jax.numpy as jnp
import numpy as np
from jax import lax
from jax.experimental import pallas as pl
from jax.experimental.pallas import tpu as pltpu
from jax.experimental.pallas import tpu_sc as plsc

N_SAMPLES = 3200000
N_RAYS = 100000
NC = 2
NS = 16
NW = NC * NS
RPW = 3128
ACC = RPW * 3
ACC_PAD = 9392
CH = 2048
GRP = CH // 16


def _extract_i32(ref, pos):
    vec = ref[pl.ds(pos, 16)]
    return vec[0]


def _sc_body(w_hbm, o_hbm, i_hbm, b_hbm, out_hbm, bnd_v, w_v, o_v, i_v, acc_v):
    wid = lax.axis_index("s") * NC + lax.axis_index("c")
    pltpu.sync_copy(b_hbm, bnd_v)
    start = _extract_i32(bnd_v, wid)
    end = _extract_i32(bnd_v, wid + 1)
    ray_lo = wid * RPW
    rpw_w = jnp.minimum(RPW, N_RAYS - ray_lo)

    zeros16 = jnp.zeros((16,), jnp.float32)

    def zero_body(k, c):
        acc_v[pl.ds(k * 16, 16)] = zeros16
        return c

    lax.fori_loop(0, ACC_PAD // 16, zero_body, None)

    base0 = start & (-8)
    nchunks = (end - base0 + CH - 1) // CH
    lane = lax.iota(jnp.int32, 16)
    l3 = lane * 3
    sh_dn = jnp.maximum(lane - 1, 0)
    sh_up = jnp.minimum(lane + 1, 15)
    lane0 = lane == 0
    lane15 = lane == 15

    def chunk_body(ci, carry):
        ub = base0 + ci * CH
        b = pl.multiple_of(jnp.minimum(ub, N_SAMPLES - CH), 8)
        d = ub - b
        pltpu.sync_copy(w_hbm.at[pl.ds(b, CH)], w_v)
        pltpu.sync_copy(o_hbm.at[pl.ds(pl.multiple_of(b * 3, 8), CH * 3)], o_v)
        pltpu.sync_copy(i_hbm.at[pl.ds(b, CH)], i_v)

        def grp_body(j, c):
            j16 = j * 16
            idx16 = i_v[pl.ds(j16, 16)]
            w16 = w_v[pl.ds(j16, 16)]
            pos_ok = (j16 + lane) >= d
            w16z = jnp.where(pos_ok, w16, 0.0)
            lv = idx16 - ray_lo
            ray_ok = (lv >= 0) & (lv < rpw_w)
            tgt0 = jnp.clip(lv, 0, RPW - 1) * 3
            prev_idx = idx16.at[sh_dn].get(mode="promise_in_bounds")
            is_start = (idx16 != prev_idx) | lane0
            sp = plsc.cummax(jnp.where(is_start, lane, 0))
            endv = jnp.where(is_start, 1, 0).at[sh_up].get(
                mode="promise_in_bounds"
            )
            end = (endv == 1) | lane15
            base_i = jnp.maximum(sp - 1, 0)
            has_prev = sp > 0
            m = end & ray_ok
            p0 = j16 * 3 + l3
            for ch in range(3):
                oc = plsc.load_gather(o_v, [p0 + ch])
                cs = plsc.cumsum(w16z * oc)
                pb = cs.at[base_i].get(mode="promise_in_bounds")
                run = cs - jnp.where(has_prev, pb, 0.0)
                plsc.addupdate_scatter(acc_v, [tgt0 + ch], run, mask=m)
            return c

        lax.fori_loop(0, GRP, grp_body, None)
        return carry

    lax.fori_loop(0, nchunks, chunk_body, None)

    pltpu.sync_copy(
        acc_v.at[pl.ds(0, ACC)],
        out_hbm.at[pl.ds(pl.multiple_of(wid * ACC, 8), ACC)],
    )


@jax.jit
def _run(w_flat, o_flat, ray_indices, bnd):
    mesh = plsc.VectorSubcoreMesh(core_axis_name="c", subcore_axis_name="s")
    k = functools.partial(
        pl.kernel,
        mesh=mesh,
        out_type=jax.ShapeDtypeStruct((NW * ACC,), jnp.float32),
        scratch_types=[
            pltpu.VMEM((48,), jnp.int32),
            pltpu.VMEM((CH,), jnp.float32),
            pltpu.VMEM((CH * 3,), jnp.float32),
            pltpu.VMEM((CH,), jnp.int32),
            pltpu.VMEM((ACC_PAD,), jnp.float32),
        ],
        compiler_params=pltpu.CompilerParams(needs_layout_passes=False),
    )(_sc_body)
    return k(w_flat, o_flat, ray_indices, bnd)


def kernel(weights, offsets, ray_indices, num_rays):
    w_flat = weights.reshape(-1)
    o_flat = offsets.reshape(-1)
    edges = jnp.arange(NW + 1, dtype=jnp.int32) * RPW
    bounds = jnp.searchsorted(ray_indices, edges).astype(jnp.int32)
    bnd = jnp.full((48,), N_SAMPLES, jnp.int32).at[: NW + 1].set(bounds)
    out = _run(w_flat, o_flat, ray_indices, bnd)
    return out[: N_RAYS * 3].reshape(N_RAYS, 3)
